# baseline (speedup 1.0000x reference)
import functools

import jax
import jax.numpy as jnp
from jax import lax
from jax.experimental import pallas as pl
from jax.experimental.pallas import tpu as pltpu
from jax.experimental.pallas import tpu_sc as plsc

U = 5000
I = 5000
E = 320000
EP = 100000
D = 128
L = 3

NC = 2
NS = 16
NW = NC * NS
LANES = 16

KE = 80
KP = 128
R = 5120
RPT = R // NS

EPAD = 327680
PPAD = 102400

CH0E, CH1E = 180, 76
CHE_MAX = max(CH0E, CH1E)
CH0S, CH1S = 74, 26
CHS_MAX = max(CH0S, CH1S)

_mesh = plsc.VectorSubcoreMesh(core_axis_name="c", subcore_axis_name="s")
_cparams = pltpu.CompilerParams(needs_layout_passes=False)
_cparams_nt = pltpu.CompilerParams(needs_layout_passes=False,
                                   use_tc_tiling_on_sc=False)


@functools.partial(
    pl.kernel,
    out_type=(
        jax.ShapeDtypeStruct((NW, R), jnp.float32),
        jax.ShapeDtypeStruct((NW, R), jnp.float32),
    ),
    mesh=_mesh,
    compiler_params=_cparams,
    scratch_types=[
        pltpu.VMEM((LANES, R), jnp.float32),
        pltpu.VMEM((R,), jnp.float32),
        pltpu.VMEM((KE,), jnp.int32),
        pltpu.VMEM((KE,), jnp.int32),
        pltpu.SemaphoreType.DMA,
        pltpu.SemaphoreType.DMA,
    ],
)
def _k_deg(src_hbm, dst_hbm, outu_hbm, outi_hbm, tab, robuf, sbufa, sbufb,
           xa, xb):
  c = lax.axis_index("c")
  s = lax.axis_index("s")
  wid = c * NS + s
  lane = lax.iota(jnp.int32, LANES)
  ones = jnp.ones((LANES,), jnp.float32)
  zeros = jnp.zeros((LANES,), jnp.float32)

  def ztab(r, _):
    for l in range(LANES):
      tab[l, pl.ds(r * LANES, LANES)] = zeros
    return 0

  def count(idx_hbm):
    pltpu.sync_copy(idx_hbm.at[wid, 0], sbufa)
    pltpu.async_copy(idx_hbm.at[wid, 1], sbufb, xb)

    def scat(buf):
      for t in range(KE // LANES):
        idxv = buf[pl.ds(t * LANES, LANES)]
        plsc.addupdate_scatter(tab, [lane, idxv], ones)

    def body(m, _):
      j = 2 * m

      @pl.when(j + 2 < CHE_MAX)
      def _():
        pltpu.async_copy(idx_hbm.at[wid, j + 2], sbufa, xa)
      scat(sbufa)
      pltpu.make_async_copy(idx_hbm.at[wid, j + 1], sbufb, xb).wait()

      @pl.when(j + 3 < CHE_MAX)
      def _():
        pltpu.async_copy(idx_hbm.at[wid, j + 3], sbufb, xb)
      scat(sbufb)

      @pl.when(j + 2 < CHE_MAX)
      def _():
        pltpu.make_async_copy(idx_hbm.at[wid, j + 2], sbufa, xa).wait()
      return 0

    lax.fori_loop(0, CHE_MAX // 2, body, 0)

  def reduce(r, _):
    acc = tab[0, pl.ds(r * LANES, LANES)]
    for l in range(1, LANES):
      acc = acc + tab[l, pl.ds(r * LANES, LANES)]
    robuf[pl.ds(r * LANES, LANES)] = acc
    return 0

  lax.fori_loop(0, R // LANES, ztab, 0)
  count(src_hbm)
  lax.fori_loop(0, R // LANES, reduce, 0)
  pltpu.sync_copy(robuf, outu_hbm.at[wid])

  lax.fori_loop(0, R // LANES, ztab, 0)
  count(dst_hbm)
  lax.fori_loop(0, R // LANES, reduce, 0)
  pltpu.sync_copy(robuf, outi_hbm.at[wid])


@functools.partial(
    pl.kernel,
    out_type=(
        jax.ShapeDtypeStruct((NC, R, D), jnp.float32),
        jax.ShapeDtypeStruct((NC, R, D), jnp.float32),
    ),
    mesh=_mesh,
    compiler_params=_cparams_nt,
    scratch_types=[
        pltpu.VMEM_SHARED((R, D), jnp.float32),
        pltpu.VMEM_SHARED((R, D), jnp.float32),
        pltpu.VMEM((KE,), jnp.int32),
        pltpu.VMEM((KE,), jnp.int32),
        pltpu.VMEM((KE,), jnp.int32),
        pltpu.VMEM((KE,), jnp.int32),
        pltpu.VMEM((KE, D // 2), jnp.int32),
        pltpu.VMEM((KE, D // 2), jnp.int32),
        pltpu.VMEM((KE, D // 2), jnp.int32),
        pltpu.VMEM((KE, D // 2), jnp.int32),
        pltpu.VMEM((KE, D), jnp.float32),
        pltpu.VMEM((KE, D), jnp.float32),
        pltpu.SemaphoreType.DMA,
        pltpu.SemaphoreType.DMA,
        pltpu.SemaphoreType.DMA,
        pltpu.SemaphoreType.DMA,
        pltpu.SemaphoreType.DMA,
        pltpu.SemaphoreType.DMA,
        pltpu.SemaphoreType.DMA,
        pltpu.SemaphoreType.DMA,
    ],
)
def _k_layer(src_hbm, dst_hbm, pu_hbm, pi_hbm, outu_hbm, outi_hbm,
             accu_sh, acci_sh,
             sbuf0, dbuf0, sbuf1, dbuf1,
             ru0, ri0, ru1, ri1, fb0, fb1,
             xu0, xi0, xu1, xi1, gu0, gi0, gu1, gi1):
  c = lax.axis_index("c")
  s = lax.axis_index("s")
  wid = c * NS + s
  sets = (
      (sbuf0, dbuf0, ru0, ri0, fb0, xu0, xi0, gu0, gi0),
      (sbuf1, dbuf1, ru1, ri1, fb1, xu1, xi1, gu1, gi1),
  )
  M = jnp.where(c == 0, CH0E // 2, CH1E // 2)

  def widen(bref, fref):
    def wrow(r, _):
      for g in range(D // 32):
        v = bref[r, pl.ds(g * LANES, LANES)]
        lo = lax.shift_left(v, 16)
        hi = jnp.bitwise_and(v, jnp.int32(-65536))
        fref[r, pl.ds(g * 32, LANES)] = plsc.bitcast(lo, jnp.float32)
        fref[r, pl.ds(g * 32 + LANES, LANES)] = plsc.bitcast(hi, jnp.float32)
      return 0

    lax.fori_loop(0, KE, wrow, 0)

  zeros = jnp.zeros((LANES,), jnp.float32)

  def zrow(r, _):
    for cc in range(D // LANES):
      fb0[r, pl.ds(cc * LANES, LANES)] = zeros
    return 0

  lax.fori_loop(0, KE, zrow, 0)
  for base in range(0, RPT, KE):
    pltpu.sync_copy(fb0, accu_sh.at[pl.ds(s * RPT + base, KE)])
    pltpu.sync_copy(fb0, acci_sh.at[pl.ds(s * RPT + base, KE)])

  pltpu.sync_copy(src_hbm.at[wid, 0], sbuf0)
  pltpu.sync_copy(dst_hbm.at[wid, 0], dbuf0)
  pltpu.async_copy(pu_hbm.at[sbuf0], ru0, gu0)
  pltpu.async_copy(pi_hbm.at[dbuf0], ri0, gi0)
  pltpu.async_copy(src_hbm.at[wid, 1], sbuf1, xu1)
  pltpu.async_copy(dst_hbm.at[wid, 1], dbuf1, xi1)
  plsc.subcore_barrier()

  def substep(m, j, cur, nxt, has_next, has_next2):
    csb, cdb, cru, cri, cfb, cxu, cxi, cgu, cgi = cur
    nsb, ndb, nru, nri, nfb, nxu, nxi, ngu, ngi = nxt

    def issue_next():
      pltpu.make_async_copy(src_hbm.at[wid, j + 1], nsb, nxu).wait()
      pltpu.make_async_copy(dst_hbm.at[wid, j + 1], ndb, nxi).wait()
      pltpu.async_copy(pu_hbm.at[nsb], nru, ngu)
      pltpu.async_copy(pi_hbm.at[ndb], nri, ngi)

    if has_next is True:
      issue_next()
    else:
      pl.when(has_next)(issue_next)

    pltpu.make_async_copy(pu_hbm.at[csb], cru, cgu).wait()
    pltpu.make_async_copy(pi_hbm.at[cdb], cri, cgi).wait()
    widen(cru, cfb)
    pltpu.sync_copy(cfb, acci_sh.at[cdb], add=True)
    widen(cri, cfb)
    pltpu.sync_copy(cfb, accu_sh.at[csb], add=True)

    def issue_idx2():
      pltpu.async_copy(src_hbm.at[wid, j + 2], csb, cxu)
      pltpu.async_copy(dst_hbm.at[wid, j + 2], cdb, cxi)

    if has_next2 is True:
      issue_idx2()
    else:
      pl.when(has_next2)(issue_idx2)

  def body(m, _):
    not_last = m < M - 1
    substep(m, 2 * m, sets[0], sets[1], True, not_last)
    substep(m, 2 * m + 1, sets[1], sets[0], not_last, not_last)
    return 0

  lax.fori_loop(0, M, body, 0)
  plsc.subcore_barrier()

  for base in range(0, RPT, KE):
    rb = pl.ds(s * RPT + base, KE)
    pltpu.sync_copy(accu_sh.at[rb], fb0)
    pltpu.sync_copy(fb0, outu_hbm.at[c, rb])
    pltpu.sync_copy(acci_sh.at[rb], fb1)
    pltpu.sync_copy(fb1, outi_hbm.at[c, rb])


def _dot_chunk(rows_a, rows_b, scr, out_vm, j):
  lane = lax.iota(jnp.int32, LANES)

  def tbody(t, _):
    for g in range(KP // LANES):
      e = g * LANES + t
      acc = rows_a[e, pl.ds(0, LANES)] * rows_b[e, pl.ds(0, LANES)]
      for cc in range(1, D // LANES):
        acc = acc + (rows_a[e, pl.ds(cc * LANES, LANES)]
                     * rows_b[e, pl.ds(cc * LANES, LANES)])
      scr[e, pl.ds(0, LANES)] = acc
    return 0

  lax.fori_loop(0, LANES, tbody, 0)
  for g in range(KP // LANES):
    rowi = lane + g * LANES
    tot = plsc.load_gather(scr, [rowi, lane])
    for c in range(1, LANES):
      col = jnp.bitwise_and(lane + c, LANES - 1)
      tot = tot + plsc.load_gather(scr, [rowi, col])
    out_vm[j, pl.ds(g * LANES, LANES)] = tot


@functools.partial(
    pl.kernel,
    out_type=jax.ShapeDtypeStruct((NW, CHS_MAX, KP), jnp.float32),
    mesh=_mesh,
    compiler_params=_cparams,
    scratch_types=[
        pltpu.VMEM((KP,), jnp.int32),
        pltpu.VMEM((KP,), jnp.int32),
        pltpu.VMEM((KP,), jnp.int32),
        pltpu.VMEM((KP,), jnp.int32),
        pltpu.VMEM((KP, D), jnp.float32),
        pltpu.VMEM((KP, D), jnp.float32),
        pltpu.VMEM((KP, D), jnp.float32),
        pltpu.VMEM((KP, D), jnp.float32),
        pltpu.VMEM((KP, LANES), jnp.float32),
        pltpu.VMEM((CHS_MAX, KP), jnp.float32),
        pltpu.SemaphoreType.DMA,
        pltpu.SemaphoreType.DMA,
        pltpu.SemaphoreType.DMA,
        pltpu.SemaphoreType.DMA,
        pltpu.SemaphoreType.DMA,
        pltpu.SemaphoreType.DMA,
        pltpu.SemaphoreType.DMA,
        pltpu.SemaphoreType.DMA,
    ],
)
def _k_score(ru_hbm, ri_hbm, uidx_hbm, iidx_hbm, out_hbm,
             abuf0, bbuf0, abuf1, bbuf1,
             rows_a0, rows_b0, rows_a1, rows_b1,
             scr, out_vm,
             xa0, xb0, xa1, xb1, ga0, gb0, ga1, gb1):
  c = lax.axis_index("c")
  s = lax.axis_index("s")
  wid = c * NS + s
  sets = (
      (abuf0, bbuf0, rows_a0, rows_b0, xa0, xb0, ga0, gb0),
      (abuf1, bbuf1, rows_a1, rows_b1, xa1, xb1, ga1, gb1),
  )
  M = jnp.where(c == 0, CH0S // 2, CH1S // 2)

  pltpu.sync_copy(uidx_hbm.at[wid, 0], abuf0)
  pltpu.sync_copy(iidx_hbm.at[wid, 0], bbuf0)
  pltpu.async_copy(ru_hbm.at[abuf0], rows_a0, ga0)
  pltpu.async_copy(ri_hbm.at[bbuf0], rows_b0, gb0)
  pltpu.async_copy(uidx_hbm.at[wid, 1], abuf1, xa1)
  pltpu.async_copy(iidx_hbm.at[wid, 1], bbuf1, xb1)

  def substep(j, cur, nxt, has_next, has_next2):
    cab, cbb, cra, crb, cxa, cxb, cga, cgb = cur
    nab, nbb, nra, nrb, nxa, nxb, nga, ngb = nxt

    def issue_next():
      pltpu.make_async_copy(uidx_hbm.at[wid, j + 1], nab, nxa).wait()
      pltpu.make_async_copy(iidx_hbm.at[wid, j + 1], nbb, nxb).wait()
      pltpu.async_copy(ru_hbm.at[nab], nra, nga)
      pltpu.async_copy(ri_hbm.at[nbb], nrb, ngb)

    if has_next is True:
      issue_next()
    else:
      pl.when(has_next)(issue_next)

    pltpu.make_async_copy(ru_hbm.at[cab], cra, cga).wait()
    pltpu.make_async_copy(ri_hbm.at[cbb], crb, cgb).wait()

    def issue_idx2():
      pltpu.async_copy(uidx_hbm.at[wid, j + 2], cab, cxa)
      pltpu.async_copy(iidx_hbm.at[wid, j + 2], cbb, cxb)

    if has_next2 is True:
      issue_idx2()
    else:
      pl.when(has_next2)(issue_idx2)

    _dot_chunk(cra, crb, scr, out_vm, j)

  def body(m, _):
    not_last = m < M - 1
    substep(2 * m, sets[0], sets[1], True, not_last)
    substep(2 * m + 1, sets[1], sets[0], not_last, not_last)
    return 0

  lax.fori_loop(0, M, body, 0)
  pltpu.sync_copy(out_vm, out_hbm.at[wid])


def _pack_core_split(a, total, fill, ch0, ch1, k):
  pad = total - a.shape[0]
  a = jnp.concatenate([a, jnp.full((pad,), fill, jnp.int32)])
  arr = a.reshape(-1, k)
  n0 = NS * ch0
  a0 = arr[:n0].reshape(NS, ch0, k)
  a1 = arr[n0:].reshape(NS, ch1, k)
  chm = max(ch0, ch1)
  if ch0 < chm:
    a0 = jnp.concatenate(
        [a0, jnp.full((NS, chm - ch0, k), fill, jnp.int32)], axis=1)
  if ch1 < chm:
    a1 = jnp.concatenate(
        [a1, jnp.full((NS, chm - ch1, k), fill, jnp.int32)], axis=1)
  return jnp.concatenate([a0, a1], axis=0)


def _pad_rows_bf16(m):
  m = jnp.concatenate(
      [m, jnp.zeros((R - m.shape[0], m.shape[1]), m.dtype)], axis=0)
  t = m.reshape(R, D // 32, 2, LANES)
  b = t.transpose(0, 1, 3, 2).reshape(R, D).astype(jnp.bfloat16)
  return lax.bitcast_convert_type(b.reshape(R, D // 2, 2), jnp.int32)


def kernel(edge_index, pos_edge_index, neg_edge_index, user_emb, item_emb):
  src = edge_index[0]
  dst = edge_index[1]
  src3 = _pack_core_split(src, EPAD, U, CH0E, CH1E, KE)
  dst3 = _pack_core_split(dst, EPAD, I, CH0E, CH1E, KE)

  du, di = _k_deg(src3, dst3)
  deg_u = jnp.sum(du, axis=0)[:U]
  deg_i = jnp.sum(di, axis=0)[:I]
  inv_su = lax.rsqrt(jnp.maximum(deg_u, 1.0))
  inv_si = lax.rsqrt(jnp.maximum(deg_i, 1.0))

  h_u, h_i = user_emb, item_emb
  res_u, res_i = user_emb, item_emb
  for l in range(L):
    pu = _pad_rows_bf16(h_u * inv_su[:, None])
    pi = _pad_rows_bf16(h_i * inv_si[:, None])
    agg_u, agg_i = _k_layer(src3, dst3, pu, pi)
    h_u = (agg_u[0] + agg_u[1])[:U] * inv_su[:, None]
    h_i = (agg_i[0] + agg_i[1])[:I] * inv_si[:, None]
    res_u = res_u + h_u * (1.0 / (l + 2))
    res_i = res_i + h_i * (1.0 / (l + 2))

  def padded(a):
    return jnp.concatenate(
        [a, jnp.zeros((PPAD - a.shape[0],), jnp.int32)])

  cu = jnp.concatenate([padded(pos_edge_index[0]), padded(neg_edge_index[0])])
  ci = jnp.concatenate([padded(pos_edge_index[1]), padded(neg_edge_index[1])])
  uidx = _pack_core_split(cu, 2 * PPAD, 0, CH0S, CH1S, KP)
  iidx = _pack_core_split(ci, 2 * PPAD, 0, CH0S, CH1S, KP)
  out = _k_score(res_u, res_i, uidx, iidx)
  flat = jnp.concatenate(
      [out[:NS, :CH0S].reshape(-1), out[NS:, :CH1S].reshape(-1)])
  pos_score = flat[:PPAD][:EP, None]
  neg_score = flat[PPAD:][:EP, None]
  return (pos_score, neg_score)


# speedup vs baseline: 1.4174x; 1.4174x over previous
import functools

import jax
import jax.numpy as jnp
from jax import lax
from jax.experimental import pallas as pl
from jax.experimental.pallas import tpu as pltpu
from jax.experimental.pallas import tpu_sc as plsc

U = 5000
I = 5000
E = 320000
EP = 100000
D = 128
L = 3

NC = 2
NS = 16
NW = NC * NS
LANES = 16

KE = 80
KP = 128
R = 5120
RPT = R // NS

EPAD = 327680
PPAD = 102400

CH0E, CH1E = 150, 106
CHE_MAX = max(CH0E, CH1E)
CH0S, CH1S = 68, 32
CHS_MAX = max(CH0S, CH1S)

_mesh = plsc.VectorSubcoreMesh(core_axis_name="c", subcore_axis_name="s")
_cparams = pltpu.CompilerParams(needs_layout_passes=False)


@functools.partial(
    pl.kernel,
    out_type=(
        jax.ShapeDtypeStruct((NW, R), jnp.float32),
        jax.ShapeDtypeStruct((NW, R), jnp.float32),
    ),
    mesh=_mesh,
    compiler_params=_cparams,
    scratch_types=[
        pltpu.VMEM((LANES, R), jnp.float32),
        pltpu.VMEM((R,), jnp.float32),
        pltpu.VMEM((KE,), jnp.int32),
        pltpu.VMEM((KE,), jnp.int32),
        pltpu.SemaphoreType.DMA,
        pltpu.SemaphoreType.DMA,
    ],
)
def _k_deg(src_hbm, dst_hbm, outu_hbm, outi_hbm, tab, robuf, sbufa, sbufb,
           xa, xb):
  c = lax.axis_index("c")
  s = lax.axis_index("s")
  wid = c * NS + s
  lane = lax.iota(jnp.int32, LANES)
  ones = jnp.ones((LANES,), jnp.float32)
  zeros = jnp.zeros((LANES,), jnp.float32)

  def ztab(r, _):
    for l in range(LANES):
      tab[l, pl.ds(r * LANES, LANES)] = zeros
    return 0

  def count(idx_hbm):
    pltpu.sync_copy(idx_hbm.at[wid, 0], sbufa)
    pltpu.async_copy(idx_hbm.at[wid, 1], sbufb, xb)

    def scat(buf):
      for t in range(KE // LANES):
        idxv = buf[pl.ds(t * LANES, LANES)]
        plsc.addupdate_scatter(tab, [lane, idxv], ones)

    def body(m, _):
      j = 2 * m

      @pl.when(j + 2 < CHE_MAX)
      def _():
        pltpu.async_copy(idx_hbm.at[wid, j + 2], sbufa, xa)
      scat(sbufa)
      pltpu.make_async_copy(idx_hbm.at[wid, j + 1], sbufb, xb).wait()

      @pl.when(j + 3 < CHE_MAX)
      def _():
        pltpu.async_copy(idx_hbm.at[wid, j + 3], sbufb, xb)
      scat(sbufb)

      @pl.when(j + 2 < CHE_MAX)
      def _():
        pltpu.make_async_copy(idx_hbm.at[wid, j + 2], sbufa, xa).wait()
      return 0

    lax.fori_loop(0, CHE_MAX // 2, body, 0)

  def reduce(r, _):
    acc = tab[0, pl.ds(r * LANES, LANES)]
    for l in range(1, LANES):
      acc = acc + tab[l, pl.ds(r * LANES, LANES)]
    robuf[pl.ds(r * LANES, LANES)] = acc
    return 0

  lax.fori_loop(0, R // LANES, ztab, 0)
  count(src_hbm)
  lax.fori_loop(0, R // LANES, reduce, 0)
  pltpu.sync_copy(robuf, outu_hbm.at[wid])

  lax.fori_loop(0, R // LANES, ztab, 0)
  count(dst_hbm)
  lax.fori_loop(0, R // LANES, reduce, 0)
  pltpu.sync_copy(robuf, outi_hbm.at[wid])


@functools.partial(
    pl.kernel,
    out_type=(
        jax.ShapeDtypeStruct((NC, R, D), jnp.float32),
        jax.ShapeDtypeStruct((NC, R, D), jnp.float32),
    ),
    mesh=_mesh,
    compiler_params=_cparams,
    scratch_types=[
        pltpu.VMEM_SHARED((R, D), jnp.float32),
        pltpu.VMEM_SHARED((R, D), jnp.float32),
        pltpu.VMEM((KE,), jnp.int32),
        pltpu.VMEM((KE,), jnp.int32),
        pltpu.VMEM((KE,), jnp.int32),
        pltpu.VMEM((KE,), jnp.int32),
        pltpu.VMEM((KE, D), jnp.float32),
        pltpu.VMEM((KE, D), jnp.float32),
        pltpu.VMEM((KE, D), jnp.float32),
        pltpu.VMEM((KE, D), jnp.float32),
        pltpu.SemaphoreType.DMA,
        pltpu.SemaphoreType.DMA,
        pltpu.SemaphoreType.DMA,
        pltpu.SemaphoreType.DMA,
        pltpu.SemaphoreType.DMA,
        pltpu.SemaphoreType.DMA,
        pltpu.SemaphoreType.DMA,
        pltpu.SemaphoreType.DMA,
    ],
)
def _k_layer(src_hbm, dst_hbm, pu_hbm, pi_hbm, outu_hbm, outi_hbm,
             accu_sh, acci_sh,
             sbuf0, dbuf0, sbuf1, dbuf1,
             ru0, ri0, ru1, ri1,
             xu0, xi0, xu1, xi1, gu0, gi0, gu1, gi1):
  c = lax.axis_index("c")
  s = lax.axis_index("s")
  wid = c * NS + s
  sets = (
      (sbuf0, dbuf0, ru0, ri0, xu0, xi0, gu0, gi0),
      (sbuf1, dbuf1, ru1, ri1, xu1, xi1, gu1, gi1),
  )
  M = jnp.where(c == 0, CH0E // 2, CH1E // 2)

  zeros = jnp.zeros((LANES,), jnp.float32)

  def zrow(r, _):
    for cc in range(D // LANES):
      ru0[r, pl.ds(cc * LANES, LANES)] = zeros
    return 0

  lax.fori_loop(0, KE, zrow, 0)
  for base in range(0, RPT, KE):
    pltpu.sync_copy(ru0, accu_sh.at[pl.ds(s * RPT + base, KE)])
    pltpu.sync_copy(ru0, acci_sh.at[pl.ds(s * RPT + base, KE)])

  pltpu.sync_copy(src_hbm.at[wid, 0], sbuf0)
  pltpu.sync_copy(dst_hbm.at[wid, 0], dbuf0)
  pltpu.async_copy(pu_hbm.at[sbuf0], ru0, gu0)
  pltpu.async_copy(pi_hbm.at[dbuf0], ri0, gi0)
  pltpu.async_copy(src_hbm.at[wid, 1], sbuf1, xu1)
  pltpu.async_copy(dst_hbm.at[wid, 1], dbuf1, xi1)
  plsc.subcore_barrier()

  def substep(m, j, cur, nxt, has_next, has_next2):
    csb, cdb, cru, cri, cxu, cxi, cgu, cgi = cur
    nsb, ndb, nru, nri, nxu, nxi, ngu, ngi = nxt

    def issue_next():
      pltpu.make_async_copy(src_hbm.at[wid, j + 1], nsb, nxu).wait()
      pltpu.make_async_copy(dst_hbm.at[wid, j + 1], ndb, nxi).wait()
      pltpu.async_copy(pu_hbm.at[nsb], nru, ngu)
      pltpu.async_copy(pi_hbm.at[ndb], nri, ngi)

    if has_next is True:
      issue_next()
    else:
      pl.when(has_next)(issue_next)

    pltpu.make_async_copy(pu_hbm.at[csb], cru, cgu).wait()
    pltpu.make_async_copy(pi_hbm.at[cdb], cri, cgi).wait()
    pltpu.sync_copy(cru, acci_sh.at[cdb], add=True)
    pltpu.sync_copy(cri, accu_sh.at[csb], add=True)

    def issue_idx2():
      pltpu.async_copy(src_hbm.at[wid, j + 2], csb, cxu)
      pltpu.async_copy(dst_hbm.at[wid, j + 2], cdb, cxi)

    if has_next2 is True:
      issue_idx2()
    else:
      pl.when(has_next2)(issue_idx2)

  def body(m, _):
    not_last = m < M - 1
    substep(m, 2 * m, sets[0], sets[1], True, not_last)
    substep(m, 2 * m + 1, sets[1], sets[0], not_last, not_last)
    return 0

  lax.fori_loop(0, M, body, 0)
  plsc.subcore_barrier()

  for base in range(0, RPT, KE):
    rb = pl.ds(s * RPT + base, KE)
    pltpu.sync_copy(accu_sh.at[rb], ru0)
    pltpu.sync_copy(ru0, outu_hbm.at[c, rb])
    pltpu.sync_copy(acci_sh.at[rb], ri0)
    pltpu.sync_copy(ri0, outi_hbm.at[c, rb])


def _dot_chunk(rows_a, rows_b, scr, out_vm, j):
  lane = lax.iota(jnp.int32, LANES)

  def tbody(t, _):
    for g in range(KP // LANES):
      e = g * LANES + t
      acc = rows_a[e, pl.ds(0, LANES)] * rows_b[e, pl.ds(0, LANES)]
      for cc in range(1, D // LANES):
        acc = acc + (rows_a[e, pl.ds(cc * LANES, LANES)]
                     * rows_b[e, pl.ds(cc * LANES, LANES)])
      scr[e, pl.ds(0, LANES)] = acc
    return 0

  lax.fori_loop(0, LANES, tbody, 0)
  for g in range(KP // LANES):
    rowi = lane + g * LANES
    tot = plsc.load_gather(scr, [rowi, lane])
    for c in range(1, LANES):
      col = jnp.bitwise_and(lane + c, LANES - 1)
      tot = tot + plsc.load_gather(scr, [rowi, col])
    out_vm[j, pl.ds(g * LANES, LANES)] = tot


@functools.partial(
    pl.kernel,
    out_type=jax.ShapeDtypeStruct((NW, CHS_MAX, KP), jnp.float32),
    mesh=_mesh,
    compiler_params=_cparams,
    scratch_types=[
        pltpu.VMEM((KP,), jnp.int32),
        pltpu.VMEM((KP,), jnp.int32),
        pltpu.VMEM((KP,), jnp.int32),
        pltpu.VMEM((KP,), jnp.int32),
        pltpu.VMEM((KP, D), jnp.float32),
        pltpu.VMEM((KP, D), jnp.float32),
        pltpu.VMEM((KP, D), jnp.float32),
        pltpu.VMEM((KP, D), jnp.float32),
        pltpu.VMEM((KP, LANES), jnp.float32),
        pltpu.VMEM((CHS_MAX, KP), jnp.float32),
        pltpu.SemaphoreType.DMA,
        pltpu.SemaphoreType.DMA,
        pltpu.SemaphoreType.DMA,
        pltpu.SemaphoreType.DMA,
        pltpu.SemaphoreType.DMA,
        pltpu.SemaphoreType.DMA,
        pltpu.SemaphoreType.DMA,
        pltpu.SemaphoreType.DMA,
    ],
)
def _k_score(ru_hbm, ri_hbm, uidx_hbm, iidx_hbm, out_hbm,
             abuf0, bbuf0, abuf1, bbuf1,
             rows_a0, rows_b0, rows_a1, rows_b1,
             scr, out_vm,
             xa0, xb0, xa1, xb1, ga0, gb0, ga1, gb1):
  c = lax.axis_index("c")
  s = lax.axis_index("s")
  wid = c * NS + s
  sets = (
      (abuf0, bbuf0, rows_a0, rows_b0, xa0, xb0, ga0, gb0),
      (abuf1, bbuf1, rows_a1, rows_b1, xa1, xb1, ga1, gb1),
  )
  M = jnp.where(c == 0, CH0S // 2, CH1S // 2)

  pltpu.sync_copy(uidx_hbm.at[wid, 0], abuf0)
  pltpu.sync_copy(iidx_hbm.at[wid, 0], bbuf0)
  pltpu.async_copy(ru_hbm.at[abuf0], rows_a0, ga0)
  pltpu.async_copy(ri_hbm.at[bbuf0], rows_b0, gb0)
  pltpu.async_copy(uidx_hbm.at[wid, 1], abuf1, xa1)
  pltpu.async_copy(iidx_hbm.at[wid, 1], bbuf1, xb1)

  def substep(j, cur, nxt, has_next, has_next2):
    cab, cbb, cra, crb, cxa, cxb, cga, cgb = cur
    nab, nbb, nra, nrb, nxa, nxb, nga, ngb = nxt

    def issue_next():
      pltpu.make_async_copy(uidx_hbm.at[wid, j + 1], nab, nxa).wait()
      pltpu.make_async_copy(iidx_hbm.at[wid, j + 1], nbb, nxb).wait()
      pltpu.async_copy(ru_hbm.at[nab], nra, nga)
      pltpu.async_copy(ri_hbm.at[nbb], nrb, ngb)

    if has_next is True:
      issue_next()
    else:
      pl.when(has_next)(issue_next)

    pltpu.make_async_copy(ru_hbm.at[cab], cra, cga).wait()
    pltpu.make_async_copy(ri_hbm.at[cbb], crb, cgb).wait()

    def issue_idx2():
      pltpu.async_copy(uidx_hbm.at[wid, j + 2], cab, cxa)
      pltpu.async_copy(iidx_hbm.at[wid, j + 2], cbb, cxb)

    if has_next2 is True:
      issue_idx2()
    else:
      pl.when(has_next2)(issue_idx2)

    _dot_chunk(cra, crb, scr, out_vm, j)

  def body(m, _):
    not_last = m < M - 1
    substep(2 * m, sets[0], sets[1], True, not_last)
    substep(2 * m + 1, sets[1], sets[0], not_last, not_last)
    return 0

  lax.fori_loop(0, M, body, 0)
  pltpu.sync_copy(out_vm, out_hbm.at[wid])


def _pack_core_split(a, total, fill, ch0, ch1, k):
  pad = total - a.shape[0]
  a = jnp.concatenate([a, jnp.full((pad,), fill, jnp.int32)])
  arr = a.reshape(-1, k)
  n0 = NS * ch0
  a0 = arr[:n0].reshape(NS, ch0, k)
  a1 = arr[n0:].reshape(NS, ch1, k)
  chm = max(ch0, ch1)
  if ch0 < chm:
    a0 = jnp.concatenate(
        [a0, jnp.full((NS, chm - ch0, k), fill, jnp.int32)], axis=1)
  if ch1 < chm:
    a1 = jnp.concatenate(
        [a1, jnp.full((NS, chm - ch1, k), fill, jnp.int32)], axis=1)
  return jnp.concatenate([a0, a1], axis=0)


def _pad_rows(m):
  return jnp.concatenate(
      [m, jnp.zeros((R - m.shape[0], m.shape[1]), m.dtype)], axis=0)


def kernel(edge_index, pos_edge_index, neg_edge_index, user_emb, item_emb):
  src = edge_index[0]
  dst = edge_index[1]
  src3 = _pack_core_split(src, EPAD, U, CH0E, CH1E, KE)
  dst3 = _pack_core_split(dst, EPAD, I, CH0E, CH1E, KE)

  du, di = _k_deg(src3, dst3)
  deg_u = jnp.sum(du, axis=0)[:U]
  deg_i = jnp.sum(di, axis=0)[:I]
  inv_su = lax.rsqrt(jnp.maximum(deg_u, 1.0))
  inv_si = lax.rsqrt(jnp.maximum(deg_i, 1.0))

  h_u, h_i = user_emb, item_emb
  res_u, res_i = user_emb, item_emb
  for l in range(L):
    pu = _pad_rows(h_u * inv_su[:, None])
    pi = _pad_rows(h_i * inv_si[:, None])
    agg_u, agg_i = _k_layer(src3, dst3, pu, pi)
    h_u = (agg_u[0] + agg_u[1])[:U] * inv_su[:, None]
    h_i = (agg_i[0] + agg_i[1])[:I] * inv_si[:, None]
    res_u = res_u + h_u * (1.0 / (l + 2))
    res_i = res_i + h_i * (1.0 / (l + 2))

  def padded(a):
    return jnp.concatenate(
        [a, jnp.zeros((PPAD - a.shape[0],), jnp.int32)])

  cu = jnp.concatenate([padded(pos_edge_index[0]), padded(neg_edge_index[0])])
  ci = jnp.concatenate([padded(pos_edge_index[1]), padded(neg_edge_index[1])])
  uidx = _pack_core_split(cu, 2 * PPAD, 0, CH0S, CH1S, KP)
  iidx = _pack_core_split(ci, 2 * PPAD, 0, CH0S, CH1S, KP)
  out = _k_score(res_u, res_i, uidx, iidx)
  flat = jnp.concatenate(
      [out[:NS, :CH0S].reshape(-1), out[NS:, :CH1S].reshape(-1)])
  pos_score = flat[:PPAD][:EP, None]
  neg_score = flat[PPAD:][:EP, None]
  return (pos_score, neg_score)
